# baseline (speedup 1.0000x reference)
import functools

import jax
import jax.numpy as jnp
from jax import lax
from jax.experimental import pallas as pl
from jax.experimental.pallas import tpu as pltpu
from jax.experimental.pallas import tpu_sc as plsc

NC, NS = 2, 16
NW = NC * NS
D = 64
V = 1000000
VFULL = (V // 128) * 128
NBLK = VFULL // 128
NI = 246

_mesh = lambda: plsc.VectorSubcoreMesh(core_axis_name="c", subcore_axis_name="s")


def _iota16():
    return lax.iota(jnp.int32, 16)


def _make_transpose():
    @functools.partial(
        pl.kernel,
        mesh=_mesh(),
        out_type=jax.ShapeDtypeStruct((V // 2, 128), jnp.float32),
        compiler_params=pltpu.CompilerParams(needs_layout_passes=False),
        scratch_types=[
            pltpu.VMEM((D, 129), jnp.float32),
            pltpu.VMEM((D, 129), jnp.float32),
            pltpu.VMEM((D, 128), jnp.float32),
            pltpu.VMEM((D, 128), jnp.float32),
            pltpu.VMEM((D, D), jnp.float32),
            pltpu.SemaphoreType.DMA,
            pltpu.SemaphoreType.DMA,
        ],
    )
    def transpose_k(tabT_hbm, tail_hbm, scr_hbm, in0, in1, ou0, ou1, tailb, isem, osem):
        wid = lax.axis_index("s") * NC + lax.axis_index("c")
        iot = _iota16()
        rows_k = [iot + 16 * m for m in range(4)]
        ins = (in0, in1)
        ous = (ou0, ou1)

        def blk_of(i):
            return wid + NW * i

        def in_dst(b):
            return ins[b].at[:, pl.ds(0, 128)]

        def transpose_block(inb, outb):
            @plsc.parallel_loop(0, D, unroll=8)
            def _(ql):
                cols0 = jnp.zeros((16,), jnp.int32) + (2 * ql)
                cols1 = cols0 + 1
                for k in range(8):
                    v = plsc.load_gather(
                        inb, [rows_k[k % 4], cols0 if k < 4 else cols1]
                    )
                    outb[ql, 16 * k : 16 * k + 16] = v

        @pl.when(blk_of(0) < NBLK)
        def _():
            pltpu.async_copy(
                tabT_hbm.at[:, pl.ds(blk_of(0) * 128, 128)], in_dst(0), isem
            )

        def pair_body(p, _):
            for b in range(2):
                i = 2 * p + b

                @pl.when(blk_of(i) < NBLK)
                def _():
                    pltpu.make_async_copy(
                        tabT_hbm.at[:, pl.ds(0, 128)], in_dst(b), isem
                    ).wait()

                    @pl.when(blk_of(i + 1) < NBLK)
                    def _():
                        pltpu.async_copy(
                            tabT_hbm.at[:, pl.ds(blk_of(i + 1) * 128, 128)],
                            in_dst(1 - b),
                            isem,
                        )

                    @pl.when(i >= 2)
                    def _():
                        pltpu.make_async_copy(
                            ous[b], scr_hbm.at[pl.ds(0, D), :], osem
                        ).wait()

                    transpose_block(ins[b], ous[b])
                    pltpu.async_copy(
                        ous[b], scr_hbm.at[pl.ds(blk_of(i) * D, D), :], osem
                    )

            return 0

        lax.fori_loop(0, NI // 2, pair_body, 0)
        for b in range(2):
            pltpu.make_async_copy(ous[b], scr_hbm.at[pl.ds(0, D), :], osem).wait()

        @pl.when(wid == NW - 1)
        def _():
            pltpu.sync_copy(tail_hbm, tailb)
            ntr = (V - VFULL) // 2

            def trow(q, _):
                for k in range(8):
                    rows = jnp.zeros((16,), jnp.int32) + (2 * q + (k // 4))
                    cols = iot + (16 * (k % 4))
                    v = plsc.load_gather(tailb, [rows, cols])
                    ou0[q, 16 * k : 16 * k + 16] = v
                return 0

            lax.fori_loop(0, ntr, trow, 0)
            pltpu.sync_copy(
                ou0.at[pl.ds(0, ntr), :],
                scr_hbm.at[pl.ds(VFULL // 2, ntr), :],
            )

    return transpose_k


def _make_gather(batch: int, seq: int):
    bw = batch // NW

    @functools.partial(
        pl.kernel,
        mesh=_mesh(),
        out_type=jax.ShapeDtypeStruct((seq, D, batch), jnp.float32),
        compiler_params=pltpu.CompilerParams(needs_layout_passes=False),
        scratch_types=[
            pltpu.VMEM((8, bw), jnp.int32),
            pltpu.VMEM((8, bw), jnp.int32),
            pltpu.VMEM((8, bw), jnp.int32),
            pltpu.VMEM((bw, 129), jnp.float32),
            pltpu.VMEM((bw, 129), jnp.float32),
            pltpu.VMEM((D, bw), jnp.float32),
            pltpu.VMEM((D, bw), jnp.float32),
            pltpu.SemaphoreType.DMA,
            pltpu.SemaphoreType.DMA,
        ],
    )
    def gather_k(idsT_hbm, scr_hbm, out_hbm, ids_v, pidx_v, off_v, bf0, bf1, bl0, bl1, gsem, wsem):
        wid = lax.axis_index("s") * NC + lax.axis_index("c")
        b0 = wid * bw
        iot = _iota16()
        rows_g = [iot + 16 * g for g in range(8)]
        bufs = (bf0, bf1)
        blks = (bl0, bl1)
        ng = bw // 16

        def s8_loop(s8i, _):
            s8 = s8i * 8
            pltpu.sync_copy(idsT_hbm.at[pl.ds(s8, 8), pl.ds(b0, bw)], ids_v)

            @plsc.parallel_loop(0, 8, unroll=2)
            def _(r):
                for k in range(ng):
                    v = ids_v[r, 16 * k : 16 * k + 16]
                    pidx_v[r, 16 * k : 16 * k + 16] = v >> 1
                    off_v[r, 16 * k : 16 * k + 16] = (v & 1) << 6

            def fire(r):
                return pltpu.async_copy(
                    scr_hbm.at[pidx_v.at[r]], bufs[r & 1].at[:, pl.ds(0, 128)], gsem
                )

            def drain_w(b):
                pltpu.make_async_copy(
                    blks[b], out_hbm.at[0, :, pl.ds(0, bw)], wsem
                ).wait()

            descs = {0: fire(0)}
            for r in range(8):
                b = r & 1
                if r < 7:
                    descs[r + 1] = fire(r + 1)
                if r >= 2:
                    drain_w(b)
                else:
                    @pl.when(s8i > 0)
                    def _():
                        drain_w(b)

                descs[r].wait()
                buf = bufs[b]
                blkb = blks[b]
                off_list = [off_v[r, 16 * g : 16 * g + 16] for g in range(ng)]

                @plsc.parallel_loop(0, D, unroll=8)
                def _(f):
                    for g in range(ng):
                        v = plsc.load_gather(buf, [rows_g[g], off_list[g] + f])
                        blkb[f, 16 * g : 16 * g + 16] = v

                pltpu.async_copy(blkb, out_hbm.at[s8 + r, :, pl.ds(b0, bw)], wsem)
            return 0

        lax.fori_loop(0, seq // 8, s8_loop, 0)
        for b in range(2):
            pltpu.make_async_copy(blks[b], out_hbm.at[0, :, pl.ds(0, bw)], wsem).wait()

    return gather_k


def kernel(input_ids, table):
    batch, seq = input_ids.shape
    ids32T = input_ids.astype(jnp.int32).T
    tabT = table.T
    tail = table[VFULL:, :]
    scr = _make_transpose()(tabT, tail)
    outT = _make_gather(batch, seq)(ids32T, scr)
    return outT.transpose(2, 0, 1)


# speedup vs baseline: 1.8344x; 1.8344x over previous
import functools

import jax
import jax.numpy as jnp
from jax import lax
from jax.experimental import pallas as pl
from jax.experimental.pallas import tpu as pltpu
from jax.experimental.pallas import tpu_sc as plsc

VOCAB_DIM = 64
NC, NS = 2, 16
NW = NC * NS
IDXW = 128
K = 5
CHUNK = K * IDXW
NBUF = 2


def _make_sc_gather(n_rows: int):
    b_per_w = n_rows // NW
    n_chunks = b_per_w // CHUNK
    n_pairs = n_chunks // NBUF
    idx_rows_per_w = b_per_w // IDXW

    mesh = plsc.VectorSubcoreMesh(core_axis_name="c", subcore_axis_name="s")

    @functools.partial(
        pl.kernel,
        mesh=mesh,
        out_type=jax.ShapeDtypeStruct((n_rows, 2 * VOCAB_DIM), jnp.float32),
        compiler_params=pltpu.CompilerParams(use_tc_tiling_on_sc=False),
        scratch_types=[
            pltpu.VMEM((NBUF, K, IDXW), jnp.int32),
            pltpu.VMEM((NBUF, CHUNK, VOCAB_DIM), jnp.float32),
            pltpu.SemaphoreType.DMA,
            pltpu.SemaphoreType.DMA,
            pltpu.SemaphoreType.DMA,
        ],
    )
    def sc_gather(ids_hbm, table_hbm, out_hbm, idx_v, rows_v, gsem, wsem0, wsem1):
        wid = lax.axis_index("s") * NC + lax.axis_index("c")
        idx_row0 = wid * idx_rows_per_w
        base0 = wid * b_per_w
        wsems = (wsem0, wsem1)

        def chunk(i, b, wait_writeback):
            rows_b = rows_v.at[b]
            idx_b = idx_v.at[b]
            if wait_writeback:
                pltpu.make_async_copy(
                    rows_b,
                    out_hbm.at[pl.ds(0, CHUNK), pl.ds(0, VOCAB_DIM)],
                    wsems[b],
                ).wait()
            pltpu.sync_copy(ids_hbm.at[pl.ds(idx_row0 + i * K, K), :], idx_b)
            copies = [
                pltpu.async_copy(
                    table_hbm.at[idx_b.at[j]],
                    rows_b.at[pl.ds(j * IDXW, IDXW), :],
                    gsem,
                )
                for j in range(K)
            ]
            for c in copies:
                c.wait()
            pltpu.async_copy(
                rows_b,
                out_hbm.at[pl.ds(base0 + i * CHUNK, CHUNK), pl.ds(0, VOCAB_DIM)],
                wsems[b],
            )

        for b in range(NBUF):
            chunk(b, b, wait_writeback=False)

        def pair_body(p, _):
            for b in range(NBUF):
                chunk(p * NBUF + b, b, wait_writeback=True)
            return 0

        lax.fori_loop(1, n_pairs, pair_body, 0)

        for b in range(NBUF):
            pltpu.make_async_copy(
                rows_v.at[b],
                out_hbm.at[pl.ds(0, CHUNK), pl.ds(0, VOCAB_DIM)],
                wsems[b],
            ).wait()

    return sc_gather


def kernel(input_ids, table):
    batch, seq = input_ids.shape
    n_rows = batch * seq
    ids = input_ids.reshape(n_rows // IDXW, IDXW).astype(jnp.int32)
    out = _make_sc_gather(n_rows)(ids, table)
    return out[:, :VOCAB_DIM].reshape(batch, seq, VOCAB_DIM)
